# Initial kernel scaffold; baseline (speedup 1.0000x reference)
import jax
import jax.numpy as jnp
from jax.experimental import pallas as pl


def kernel(x, W_stem, g_stem, b_stem, Wp0_0, gp0_0, bp0_0, Wp1_0, gp1_0, bp1_0, Wo0_0, Wo1_0, Wo2_0, Wp0_1, gp0_1, bp0_1, Wp1_1, gp1_1, bp1_1, Wo0_1, Wo1_1, Wo2_1, Wc, bc, edge_index):
    raise NotImplementedError("write your pallas kernel here")


# speedup vs baseline: 7.1910x; 7.1910x over previous
import functools

import jax
import jax.numpy as jnp
from jax import lax
from jax.experimental import pallas as pl
from jax.experimental.pallas import tpu as pltpu
from jax.experimental.pallas import tpu_sc as plsc

N = 10000
E = 320000
C = 128
NS = 16
NC = 2
K = 128
NCHUNK = 160
BI = 16
NB = NCHUNK // BI
E_PAD = NS * NCHUNK * K
N_PAD = 10240
ROWS_PER_SUB = N_PAD // NS


def _sc_agg_body(with_count, h0, h1, srcr, dstr, *rest):
    if with_count:
        out0, out1, cnt_out, src_v, dst_v, rows_a, rows_b, ones_v, ztile, zvec, acc, cntacc, sem_a, sem_b = rest
    else:
        out0, out1, src_v, dst_v, rows_a, rows_b, ztile, zvec, acc, sem_a, sem_b = rest
    c = lax.axis_index("c")
    s = lax.axis_index("s")

    for i in range(16):
        for j in range(C // 16):
            ztile[i, pl.ds(j * 16, 16)] = jnp.zeros((16,), jnp.float32)
    for j in range(ROWS_PER_SUB // 16):
        zvec[pl.ds(j * 16, 16)] = jnp.zeros((16,), jnp.float32)
    if with_count:
        for j in range(K // 16):
            ones_v[pl.ds(j * 16, 16)] = jnp.ones((16,), jnp.float32)

    def zero_body(i, _):
        pltpu.sync_copy(ztile, acc.at[pl.ds(s * ROWS_PER_SUB + i * 16, 16)])
        return 0
    lax.fori_loop(0, ROWS_PER_SUB // 16, zero_body, 0)
    if with_count:
        @pl.when(c == 0)
        def _():
            pltpu.sync_copy(zvec, cntacc.at[pl.ds(s * ROWS_PER_SUB, ROWS_PER_SUB)])

    plsc.subcore_barrier()

    def issue(j, buf, sem):
        @pl.when(c == 0)
        def _():
            pltpu.async_copy(h0.at[src_v.at[j]], buf, sem)
        @pl.when(c == 1)
        def _():
            pltpu.async_copy(h1.at[src_v.at[j]], buf, sem)

    def wait(buf, sem):
        pltpu.make_async_copy(h0.at[src_v.at[0]], buf, sem).wait()

    def scatter(j, buf):
        pltpu.sync_copy(buf, acc.at[dst_v.at[j]], add=True)
        if with_count:
            @pl.when(c == 0)
            def _():
                pltpu.sync_copy(ones_v, cntacc.at[dst_v.at[j]], add=True)

    def block_body(b, _):
        pltpu.sync_copy(srcr.at[s, pl.ds(b * BI, BI)], src_v)
        pltpu.sync_copy(dstr.at[s, pl.ds(b * BI, BI)], dst_v)
        issue(0, rows_a, sem_a)

        def loop_body(i, _):
            j = 2 * i
            issue(j + 1, rows_b, sem_b)
            wait(rows_a, sem_a)
            scatter(j, rows_a)
            @pl.when(i < BI // 2 - 1)
            def _():
                issue(j + 2, rows_a, sem_a)
            wait(rows_b, sem_b)
            scatter(j + 1, rows_b)
            return 0
        lax.fori_loop(0, BI // 2, loop_body, 0)
        return 0
    lax.fori_loop(0, NB, block_body, 0)

    plsc.subcore_barrier()
    @pl.when(c == 0)
    def _():
        pltpu.sync_copy(acc.at[pl.ds(s * ROWS_PER_SUB, ROWS_PER_SUB)],
                        out0.at[pl.ds(s * ROWS_PER_SUB, ROWS_PER_SUB)])
        if with_count:
            pltpu.sync_copy(cntacc.at[pl.ds(s * ROWS_PER_SUB, ROWS_PER_SUB)],
                            cnt_out.at[pl.ds(s * ROWS_PER_SUB, ROWS_PER_SUB)])
    @pl.when(c == 1)
    def _():
        pltpu.sync_copy(acc.at[pl.ds(s * ROWS_PER_SUB, ROWS_PER_SUB)],
                        out1.at[pl.ds(s * ROWS_PER_SUB, ROWS_PER_SUB)])


@functools.cache
def _make_sc_agg(with_count):
    mesh = plsc.VectorSubcoreMesh(core_axis_name="c", subcore_axis_name="s")
    out_type = [jax.ShapeDtypeStruct((N_PAD, C), jnp.float32),
                jax.ShapeDtypeStruct((N_PAD, C), jnp.float32)]
    scratch = [
        pltpu.VMEM((BI, K), jnp.int32),
        pltpu.VMEM((BI, K), jnp.int32),
        pltpu.VMEM((K, C), jnp.float32),
        pltpu.VMEM((K, C), jnp.float32),
    ]
    if with_count:
        out_type.append(jax.ShapeDtypeStruct((N_PAD,), jnp.float32))
        scratch.append(pltpu.VMEM((K,), jnp.float32))
    scratch.append(pltpu.VMEM((16, C), jnp.float32))
    scratch.append(pltpu.VMEM((ROWS_PER_SUB,), jnp.float32))
    scratch.append(pltpu.VMEM_SHARED((N_PAD, C), jnp.float32))
    if with_count:
        scratch.append(pltpu.VMEM_SHARED((N_PAD,), jnp.float32))
    scratch.append(pltpu.SemaphoreType.DMA)
    scratch.append(pltpu.SemaphoreType.DMA)
    return pl.kernel(
        functools.partial(_sc_agg_body, with_count),
        out_type=tuple(out_type),
        mesh=mesh,
        scratch_types=tuple(scratch),
    )


def _segment_pair(h0, h1, srcr, dstr, with_count):
    if with_count:
        a0, a1, cnt = _make_sc_agg(True)(h0, h1, srcr, dstr)
        return a0[:N], a1[:N], cnt[:N].reshape(N, 1)
    a0, a1 = _make_sc_agg(False)(h0, h1, srcr, dstr)
    return a0[:N], a1[:N]


BLK = 1000
GRID = N // BLK
_EPS = 1e-5


def _colstats(st_ref, y, idx, row):
    @pl.when(idx == 0)
    def _():
        st_ref[...] = jnp.zeros(st_ref.shape, st_ref.dtype)
    del row
    st_ref[0:1, :] += jnp.sum(y, axis=0, keepdims=True)
    st_ref[1:2, :] += jnp.sum(y * y, axis=0, keepdims=True)


def _norm(z, st, g, b):
    mu = st[0:1, :] * (1.0 / N)
    var = st[1:2, :] * (1.0 / N) - mu * mu
    inv = lax.rsqrt(var + _EPS)
    return (z - mu) * (inv * g) + b


def _stem_kernel(x_ref, w_ref, y_ref, st_ref):
    i = pl.program_id(0)
    y = jnp.dot(x_ref[...], w_ref[...], preferred_element_type=jnp.float32)
    y_ref[...] = y
    _colstats(st_ref, y, i, 0)


def _proj0_kernel(y_ref, st_ref, g_ref, b_ref, w0_ref, w1_ref,
                  s_ref, z0_ref, st0_ref, z1_ref, st1_ref):
    i = pl.program_id(0)
    s = _norm(y_ref[...], st_ref[...], g_ref[...], b_ref[...])
    s_ref[...] = s
    z0 = jnp.dot(s, w0_ref[...], preferred_element_type=jnp.float32)
    z1 = jnp.dot(s, w1_ref[...], preferred_element_type=jnp.float32)
    z0_ref[...] = z0
    z1_ref[...] = z1
    _colstats(st0_ref, z0, i, 0)
    _colstats(st1_ref, z1, i, 0)


def _norm2_kernel(z0_ref, st0_ref, g0_ref, b0_ref, z1_ref, st1_ref, g1_ref, b1_ref,
                  o0_ref, o1_ref):
    o0_ref[...] = jnp.maximum(_norm(z0_ref[...], st0_ref[...], g0_ref[...], b0_ref[...]), 0.0)
    o1_ref[...] = jnp.maximum(_norm(z1_ref[...], st1_ref[...], g1_ref[...], b1_ref[...]), 0.0)


def _cell0_proj1_kernel(s0_ref, a0_ref, s1_ref, a1_ref, cnt_ref,
                        wo0_ref, wo1_ref, wo2_ref, s_ref, wp0_ref, wp1a_ref, wp1b_ref,
                        z0_ref, st0_ref, z1_ref, st1_ref):
    i = pl.program_id(0)
    rc = 1.0 / jnp.maximum(cnt_ref[...], 1.0)
    t0 = s0_ref[...] + a0_ref[...] * rc
    t1 = s1_ref[...] + a1_ref[...] * rc
    s2 = (jnp.maximum(jnp.dot(t0, wo0_ref[...], preferred_element_type=jnp.float32), 0.0)
          + jnp.maximum(jnp.dot(t1, wo1_ref[...], preferred_element_type=jnp.float32), 0.0))
    s3 = jnp.maximum(jnp.dot(t1, wo2_ref[...], preferred_element_type=jnp.float32), 0.0) + s2
    z0 = jnp.dot(s_ref[...], wp0_ref[...], preferred_element_type=jnp.float32)
    z1 = (jnp.dot(s2, wp1a_ref[...], preferred_element_type=jnp.float32)
          + jnp.dot(s3, wp1b_ref[...], preferred_element_type=jnp.float32))
    z0_ref[...] = z0
    z1_ref[...] = z1
    _colstats(st0_ref, z0, i, 0)
    _colstats(st1_ref, z1, i, 0)


def _cell1_cls_kernel(s0_ref, a0_ref, s1_ref, a1_ref, cnt_ref,
                      wo0_ref, wo1_ref, wo2_ref, wc0_ref, wca_ref, wcb_ref, bc_ref,
                      out_ref):
    rc = 1.0 / jnp.maximum(cnt_ref[...], 1.0)
    t0 = s0_ref[...] + a0_ref[...] * rc
    t1 = s1_ref[...] + a1_ref[...] * rc
    s2 = (jnp.maximum(jnp.dot(t0, wo0_ref[...], preferred_element_type=jnp.float32), 0.0)
          + jnp.maximum(jnp.dot(t1, wo1_ref[...], preferred_element_type=jnp.float32), 0.0))
    s3 = jnp.maximum(jnp.dot(t1, wo2_ref[...], preferred_element_type=jnp.float32), 0.0) + s2
    pooled = (jnp.sum(s2, axis=1, keepdims=True)
              + jnp.sum(s3, axis=1, keepdims=True)) * (1.0 / (2 * C))
    out_ref[...] = (jnp.dot(s2, wca_ref[...], preferred_element_type=jnp.float32)
                    + jnp.dot(s3, wcb_ref[...], preferred_element_type=jnp.float32)
                    + pooled * wc0_ref[...] + bc_ref[...])


def _row_spec(d):
    return pl.BlockSpec((BLK, d), lambda i: (i, 0))


def _full_spec(a, b):
    return pl.BlockSpec((a, b), lambda i: (0, 0))


def _f32(*shape):
    return jax.ShapeDtypeStruct(shape, jnp.float32)


def _stem(x, w):
    return pl.pallas_call(
        _stem_kernel, grid=(GRID,),
        in_specs=[_row_spec(x.shape[1]), _full_spec(*w.shape)],
        out_specs=[_row_spec(w.shape[1]), _full_spec(8, w.shape[1])],
        out_shape=[_f32(N, w.shape[1]), _f32(8, w.shape[1])],
    )(x, w)


def _proj0(y, st, g, b, w0, w1):
    d = y.shape[1]
    return pl.pallas_call(
        _proj0_kernel, grid=(GRID,),
        in_specs=[_row_spec(d), _full_spec(8, d), _full_spec(1, d), _full_spec(1, d),
                  _full_spec(d, C), _full_spec(d, C)],
        out_specs=[_row_spec(d), _row_spec(C), _full_spec(8, C),
                   _row_spec(C), _full_spec(8, C)],
        out_shape=[_f32(N, d), _f32(N, C), _f32(8, C), _f32(N, C), _f32(8, C)],
    )(y, st, g, b, w0, w1)


def _norm2(z0, st0, g0, b0, z1, st1, g1, b1):
    return pl.pallas_call(
        _norm2_kernel, grid=(GRID,),
        in_specs=[_row_spec(C), _full_spec(8, C), _full_spec(1, C), _full_spec(1, C),
                  _row_spec(C), _full_spec(8, C), _full_spec(1, C), _full_spec(1, C)],
        out_specs=[_row_spec(C), _row_spec(C)],
        out_shape=[_f32(N, C), _f32(N, C)],
    )(z0, st0, g0, b0, z1, st1, g1, b1)


def _cell0_proj1(s0, a0, s1, a1, cnt, wo0, wo1, wo2, s, wp0, wp1a, wp1b):
    d = s.shape[1]
    return pl.pallas_call(
        _cell0_proj1_kernel, grid=(GRID,),
        in_specs=[_row_spec(C), _row_spec(C), _row_spec(C), _row_spec(C),
                  _row_spec(1),
                  _full_spec(C, C), _full_spec(C, C), _full_spec(C, C),
                  _row_spec(d), _full_spec(d, C), _full_spec(C, C), _full_spec(C, C)],
        out_specs=[_row_spec(C), _full_spec(8, C), _row_spec(C), _full_spec(8, C)],
        out_shape=[_f32(N, C), _f32(8, C), _f32(N, C), _f32(8, C)],
    )(s0, a0, s1, a1, cnt, wo0, wo1, wo2, s, wp0, wp1a, wp1b)


def _cell1_cls(s0, a0, s1, a1, cnt, wo0, wo1, wo2, wc0, wca, wcb, bc):
    nc = wca.shape[1]
    return pl.pallas_call(
        _cell1_cls_kernel, grid=(GRID,),
        in_specs=[_row_spec(C), _row_spec(C), _row_spec(C), _row_spec(C),
                  _row_spec(1),
                  _full_spec(C, C), _full_spec(C, C), _full_spec(C, C),
                  _full_spec(1, nc), _full_spec(C, nc), _full_spec(C, nc),
                  _full_spec(1, nc)],
        out_specs=_row_spec(nc),
        out_shape=_f32(N, nc),
    )(s0, a0, s1, a1, cnt, wo0, wo1, wo2, wc0, wca, wcb, bc)


def kernel(x, W_stem, g_stem, b_stem, Wp0_0, gp0_0, bp0_0, Wp1_0, gp1_0, bp1_0,
           Wo0_0, Wo1_0, Wo2_0, Wp0_1, gp0_1, bp0_1, Wp1_1, gp1_1, bp1_1,
           Wo0_1, Wo1_1, Wo2_1, Wc, bc, edge_index):
    src = edge_index[0]
    dst = edge_index[1]
    pad = E_PAD - E
    fidx = jnp.arange(pad, dtype=jnp.int32)
    srcr = jnp.concatenate([src, (fidx * 37) % N]).reshape(NS, NCHUNK, K)
    dstr = jnp.concatenate([dst, N + fidx % (N_PAD - N)]).reshape(NS, NCHUNK, K)

    r = lambda v: v.reshape(1, -1)

    y, sty = _stem(x, W_stem)
    s, z0, st0, z1, st1 = _proj0(y, sty, r(g_stem), r(b_stem), Wp0_0, Wp1_0)
    s0p, s1p = _norm2(z0, st0, r(gp0_0), r(bp0_0), z1, st1, r(gp1_0), r(bp1_0))
    a0, a1, cnt = _segment_pair(s0p, s1p, srcr, dstr, True)
    z0b, st0b, z1b, st1b = _cell0_proj1(
        s0p, a0, s1p, a1, cnt, Wo0_0, Wo1_0, Wo2_0, s, Wp0_1,
        Wp1_1[:C], Wp1_1[C:])
    s0q, s1q = _norm2(z0b, st0b, r(gp0_1), r(bp0_1), z1b, st1b, r(gp1_1), r(bp1_1))
    b0, b1 = _segment_pair(s0q, s1q, srcr, dstr, False)
    logits = _cell1_cls(
        s0q, b0, s1q, b1, cnt, Wo0_1, Wo1_1, Wo2_1,
        r(Wc[0]), Wc[1:1 + C], Wc[1 + C:], r(bc))
    return logits


# speedup vs baseline: 7.6526x; 1.0642x over previous
import functools

import jax
import jax.numpy as jnp
from jax import lax
from jax.experimental import pallas as pl
from jax.experimental.pallas import tpu as pltpu
from jax.experimental.pallas import tpu_sc as plsc

N = 10000
E = 320000
C = 128
NS = 16
NC = 2
K = 128
NCHUNK = 160
BI = 40
NB = NCHUNK // BI
E_PAD = NS * NCHUNK * K
N_PAD = 10240
ROWS_PER_SUB = N_PAD // NS


def _sc_agg_body(with_count, h0, h1, srcr, dstr, *rest):
    if with_count:
        out0, out1, cnt_out, src_v, dst_v, rows_a, rows_b, ones_v, ztile, zvec, acc, cntacc, sem_a, sem_b = rest
    else:
        out0, out1, src_v, dst_v, rows_a, rows_b, ztile, zvec, acc, sem_a, sem_b = rest
    c = lax.axis_index("c")
    s = lax.axis_index("s")

    for i in range(16):
        for j in range(C // 16):
            ztile[i, pl.ds(j * 16, 16)] = jnp.zeros((16,), jnp.float32)
    for j in range(ROWS_PER_SUB // 16):
        zvec[pl.ds(j * 16, 16)] = jnp.zeros((16,), jnp.float32)
    if with_count:
        for j in range(K // 16):
            ones_v[pl.ds(j * 16, 16)] = jnp.ones((16,), jnp.float32)

    def zero_body(i, _):
        pltpu.sync_copy(ztile, acc.at[pl.ds(s * ROWS_PER_SUB + i * 16, 16)])
        return 0
    lax.fori_loop(0, ROWS_PER_SUB // 16, zero_body, 0)
    if with_count:
        @pl.when(c == 0)
        def _():
            pltpu.sync_copy(zvec, cntacc.at[pl.ds(s * ROWS_PER_SUB, ROWS_PER_SUB)])

    plsc.subcore_barrier()

    def issue(j, buf, sem):
        @pl.when(c == 0)
        def _():
            pltpu.async_copy(h0.at[src_v.at[j]], buf, sem)
        @pl.when(c == 1)
        def _():
            pltpu.async_copy(h1.at[src_v.at[j]], buf, sem)

    def wait(buf, sem):
        pltpu.make_async_copy(h0.at[src_v.at[0]], buf, sem).wait()

    def scatter(j, buf):
        pltpu.sync_copy(buf, acc.at[dst_v.at[j]], add=True)
        if with_count:
            @pl.when(c == 0)
            def _():
                pltpu.sync_copy(ones_v, cntacc.at[dst_v.at[j]], add=True)

    def block_body(b, _):
        pltpu.sync_copy(srcr.at[s, pl.ds(b * BI, BI)], src_v)
        pltpu.sync_copy(dstr.at[s, pl.ds(b * BI, BI)], dst_v)
        issue(0, rows_a, sem_a)

        def loop_body(i, _):
            j = 2 * i
            issue(j + 1, rows_b, sem_b)
            wait(rows_a, sem_a)
            scatter(j, rows_a)
            @pl.when(i < BI // 2 - 1)
            def _():
                issue(j + 2, rows_a, sem_a)
            wait(rows_b, sem_b)
            scatter(j + 1, rows_b)
            return 0
        lax.fori_loop(0, BI // 2, loop_body, 0)
        return 0
    lax.fori_loop(0, NB, block_body, 0)

    plsc.subcore_barrier()
    @pl.when(c == 0)
    def _():
        pltpu.sync_copy(acc.at[pl.ds(s * ROWS_PER_SUB, ROWS_PER_SUB)],
                        out0.at[pl.ds(s * ROWS_PER_SUB, ROWS_PER_SUB)])
        if with_count:
            pltpu.sync_copy(cntacc.at[pl.ds(s * ROWS_PER_SUB, ROWS_PER_SUB)],
                            cnt_out.at[pl.ds(s * ROWS_PER_SUB, ROWS_PER_SUB)])
    @pl.when(c == 1)
    def _():
        pltpu.sync_copy(acc.at[pl.ds(s * ROWS_PER_SUB, ROWS_PER_SUB)],
                        out1.at[pl.ds(s * ROWS_PER_SUB, ROWS_PER_SUB)])


@functools.cache
def _make_sc_agg(with_count):
    mesh = plsc.VectorSubcoreMesh(core_axis_name="c", subcore_axis_name="s")
    out_type = [jax.ShapeDtypeStruct((N_PAD, C), jnp.float32),
                jax.ShapeDtypeStruct((N_PAD, C), jnp.float32)]
    scratch = [
        pltpu.VMEM((BI, K), jnp.int32),
        pltpu.VMEM((BI, K), jnp.int32),
        pltpu.VMEM((K, C), jnp.float32),
        pltpu.VMEM((K, C), jnp.float32),
    ]
    if with_count:
        out_type.append(jax.ShapeDtypeStruct((N_PAD,), jnp.float32))
        scratch.append(pltpu.VMEM((K,), jnp.float32))
    scratch.append(pltpu.VMEM((16, C), jnp.float32))
    scratch.append(pltpu.VMEM((ROWS_PER_SUB,), jnp.float32))
    scratch.append(pltpu.VMEM_SHARED((N_PAD, C), jnp.float32))
    if with_count:
        scratch.append(pltpu.VMEM_SHARED((N_PAD,), jnp.float32))
    scratch.append(pltpu.SemaphoreType.DMA)
    scratch.append(pltpu.SemaphoreType.DMA)
    return pl.kernel(
        functools.partial(_sc_agg_body, with_count),
        out_type=tuple(out_type),
        mesh=mesh,
        scratch_types=tuple(scratch),
    )


def _segment_pair(h0, h1, srcr, dstr, with_count):
    if with_count:
        a0, a1, cnt = _make_sc_agg(True)(h0, h1, srcr, dstr)
        return a0[:N], a1[:N], cnt[:N].reshape(N, 1)
    a0, a1 = _make_sc_agg(False)(h0, h1, srcr, dstr)
    return a0[:N], a1[:N]


BLK = 1000
GRID = N // BLK
_EPS = 1e-5


def _colstats(st_ref, y, idx, row):
    @pl.when(idx == 0)
    def _():
        st_ref[...] = jnp.zeros(st_ref.shape, st_ref.dtype)
    del row
    st_ref[0:1, :] += jnp.sum(y, axis=0, keepdims=True)
    st_ref[1:2, :] += jnp.sum(y * y, axis=0, keepdims=True)


def _norm(z, st, g, b):
    mu = st[0:1, :] * (1.0 / N)
    var = st[1:2, :] * (1.0 / N) - mu * mu
    inv = lax.rsqrt(var + _EPS)
    return (z - mu) * (inv * g) + b


def _stem_kernel(x_ref, w_ref, y_ref, st_ref):
    i = pl.program_id(0)
    y = jnp.dot(x_ref[...], w_ref[...], preferred_element_type=jnp.float32)
    y_ref[...] = y
    _colstats(st_ref, y, i, 0)


def _proj0_kernel(y_ref, st_ref, g_ref, b_ref, w0_ref, w1_ref,
                  z0_ref, st0_ref, z1_ref, st1_ref):
    i = pl.program_id(0)
    s = _norm(y_ref[...], st_ref[...], g_ref[...], b_ref[...])
    z0 = jnp.dot(s, w0_ref[...], preferred_element_type=jnp.float32)
    z1 = jnp.dot(s, w1_ref[...], preferred_element_type=jnp.float32)
    z0_ref[...] = z0
    z1_ref[...] = z1
    _colstats(st0_ref, z0, i, 0)
    _colstats(st1_ref, z1, i, 0)


def _norm2_kernel(z0_ref, st0_ref, g0_ref, b0_ref, z1_ref, st1_ref, g1_ref, b1_ref,
                  o0_ref, o1_ref):
    o0_ref[...] = jnp.maximum(_norm(z0_ref[...], st0_ref[...], g0_ref[...], b0_ref[...]), 0.0)
    o1_ref[...] = jnp.maximum(_norm(z1_ref[...], st1_ref[...], g1_ref[...], b1_ref[...]), 0.0)


def _cell0_proj1_kernel(s0_ref, a0_ref, s1_ref, a1_ref, cnt_ref,
                        wo0_ref, wo1_ref, wo2_ref, y_ref, sty_ref, gs_ref, bs_ref,
                        wp0_ref, wp1a_ref, wp1b_ref,
                        z0_ref, st0_ref, z1_ref, st1_ref):
    i = pl.program_id(0)
    rc = 1.0 / jnp.maximum(cnt_ref[...], 1.0)
    t0 = s0_ref[...] + a0_ref[...] * rc
    t1 = s1_ref[...] + a1_ref[...] * rc
    s2 = (jnp.maximum(jnp.dot(t0, wo0_ref[...], preferred_element_type=jnp.float32), 0.0)
          + jnp.maximum(jnp.dot(t1, wo1_ref[...], preferred_element_type=jnp.float32), 0.0))
    s3 = jnp.maximum(jnp.dot(t1, wo2_ref[...], preferred_element_type=jnp.float32), 0.0) + s2
    s = _norm(y_ref[...], sty_ref[...], gs_ref[...], bs_ref[...])
    z0 = jnp.dot(s, wp0_ref[...], preferred_element_type=jnp.float32)
    z1 = (jnp.dot(s2, wp1a_ref[...], preferred_element_type=jnp.float32)
          + jnp.dot(s3, wp1b_ref[...], preferred_element_type=jnp.float32))
    z0_ref[...] = z0
    z1_ref[...] = z1
    _colstats(st0_ref, z0, i, 0)
    _colstats(st1_ref, z1, i, 0)


def _cell1_cls_kernel(s0_ref, a0_ref, s1_ref, a1_ref, cnt_ref,
                      wo0_ref, wo1_ref, wo2_ref, wc0_ref, wca_ref, wcb_ref, bc_ref,
                      out_ref):
    rc = 1.0 / jnp.maximum(cnt_ref[...], 1.0)
    t0 = s0_ref[...] + a0_ref[...] * rc
    t1 = s1_ref[...] + a1_ref[...] * rc
    s2 = (jnp.maximum(jnp.dot(t0, wo0_ref[...], preferred_element_type=jnp.float32), 0.0)
          + jnp.maximum(jnp.dot(t1, wo1_ref[...], preferred_element_type=jnp.float32), 0.0))
    s3 = jnp.maximum(jnp.dot(t1, wo2_ref[...], preferred_element_type=jnp.float32), 0.0) + s2
    pooled = (jnp.sum(s2, axis=1, keepdims=True)
              + jnp.sum(s3, axis=1, keepdims=True)) * (1.0 / (2 * C))
    out_ref[...] = (jnp.dot(s2, wca_ref[...], preferred_element_type=jnp.float32)
                    + jnp.dot(s3, wcb_ref[...], preferred_element_type=jnp.float32)
                    + pooled * wc0_ref[...] + bc_ref[...])


def _row_spec(d):
    return pl.BlockSpec((BLK, d), lambda i: (i, 0))


def _full_spec(a, b):
    return pl.BlockSpec((a, b), lambda i: (0, 0))


def _f32(*shape):
    return jax.ShapeDtypeStruct(shape, jnp.float32)


def _stem(x, w):
    return pl.pallas_call(
        _stem_kernel, grid=(GRID,),
        in_specs=[_row_spec(x.shape[1]), _full_spec(*w.shape)],
        out_specs=[_row_spec(w.shape[1]), _full_spec(8, w.shape[1])],
        out_shape=[_f32(N, w.shape[1]), _f32(8, w.shape[1])],
    )(x, w)


def _proj0(y, st, g, b, w0, w1):
    d = y.shape[1]
    return pl.pallas_call(
        _proj0_kernel, grid=(GRID,),
        in_specs=[_row_spec(d), _full_spec(8, d), _full_spec(1, d), _full_spec(1, d),
                  _full_spec(d, C), _full_spec(d, C)],
        out_specs=[_row_spec(C), _full_spec(8, C),
                   _row_spec(C), _full_spec(8, C)],
        out_shape=[_f32(N, C), _f32(8, C), _f32(N, C), _f32(8, C)],
    )(y, st, g, b, w0, w1)


def _norm2(z0, st0, g0, b0, z1, st1, g1, b1):
    return pl.pallas_call(
        _norm2_kernel, grid=(GRID,),
        in_specs=[_row_spec(C), _full_spec(8, C), _full_spec(1, C), _full_spec(1, C),
                  _row_spec(C), _full_spec(8, C), _full_spec(1, C), _full_spec(1, C)],
        out_specs=[_row_spec(C), _row_spec(C)],
        out_shape=[_f32(N, C), _f32(N, C)],
    )(z0, st0, g0, b0, z1, st1, g1, b1)


def _cell0_proj1(s0, a0, s1, a1, cnt, wo0, wo1, wo2, y, sty, gs, bs, wp0, wp1a, wp1b):
    d = y.shape[1]
    return pl.pallas_call(
        _cell0_proj1_kernel, grid=(GRID,),
        in_specs=[_row_spec(C), _row_spec(C), _row_spec(C), _row_spec(C),
                  _row_spec(1),
                  _full_spec(C, C), _full_spec(C, C), _full_spec(C, C),
                  _row_spec(d), _full_spec(8, d), _full_spec(1, d), _full_spec(1, d),
                  _full_spec(d, C), _full_spec(C, C), _full_spec(C, C)],
        out_specs=[_row_spec(C), _full_spec(8, C), _row_spec(C), _full_spec(8, C)],
        out_shape=[_f32(N, C), _f32(8, C), _f32(N, C), _f32(8, C)],
    )(s0, a0, s1, a1, cnt, wo0, wo1, wo2, y, sty, gs, bs, wp0, wp1a, wp1b)


def _cell1_cls(s0, a0, s1, a1, cnt, wo0, wo1, wo2, wc0, wca, wcb, bc):
    nc = wca.shape[1]
    return pl.pallas_call(
        _cell1_cls_kernel, grid=(GRID,),
        in_specs=[_row_spec(C), _row_spec(C), _row_spec(C), _row_spec(C),
                  _row_spec(1),
                  _full_spec(C, C), _full_spec(C, C), _full_spec(C, C),
                  _full_spec(1, nc), _full_spec(C, nc), _full_spec(C, nc),
                  _full_spec(1, nc)],
        out_specs=_row_spec(nc),
        out_shape=_f32(N, nc),
    )(s0, a0, s1, a1, cnt, wo0, wo1, wo2, wc0, wca, wcb, bc)


def kernel(x, W_stem, g_stem, b_stem, Wp0_0, gp0_0, bp0_0, Wp1_0, gp1_0, bp1_0,
           Wo0_0, Wo1_0, Wo2_0, Wp0_1, gp0_1, bp0_1, Wp1_1, gp1_1, bp1_1,
           Wo0_1, Wo1_1, Wo2_1, Wc, bc, edge_index):
    src = edge_index[0]
    dst = edge_index[1]
    pad = E_PAD - E
    fidx = jnp.arange(pad, dtype=jnp.int32)
    srcr = jnp.concatenate([src, (fidx * 37) % N]).reshape(NS, NCHUNK, K)
    dstr = jnp.concatenate([dst, N + fidx % (N_PAD - N)]).reshape(NS, NCHUNK, K)

    r = lambda v: v.reshape(1, -1)

    y, sty = _stem(x, W_stem)
    z0, st0, z1, st1 = _proj0(y, sty, r(g_stem), r(b_stem), Wp0_0, Wp1_0)
    s0p, s1p = _norm2(z0, st0, r(gp0_0), r(bp0_0), z1, st1, r(gp1_0), r(bp1_0))
    a0, a1, cnt = _segment_pair(s0p, s1p, srcr, dstr, True)
    z0b, st0b, z1b, st1b = _cell0_proj1(
        s0p, a0, s1p, a1, cnt, Wo0_0, Wo1_0, Wo2_0, y, sty, r(g_stem), r(b_stem),
        Wp0_1, Wp1_1[:C], Wp1_1[C:])
    s0q, s1q = _norm2(z0b, st0b, r(gp0_1), r(bp0_1), z1b, st1b, r(gp1_1), r(bp1_1))
    b0, b1 = _segment_pair(s0q, s1q, srcr, dstr, False)
    logits = _cell1_cls(
        s0q, b0, s1q, b1, cnt, Wo0_1, Wo1_1, Wo2_1,
        r(Wc[0]), Wc[1:1 + C], Wc[1 + C:], r(bc))
    return logits


# speedup vs baseline: 7.9625x; 1.0405x over previous
import functools

import jax
import jax.numpy as jnp
from jax import lax
from jax.experimental import pallas as pl
from jax.experimental.pallas import tpu as pltpu
from jax.experimental.pallas import tpu_sc as plsc

N = 10000
E = 320000
C = 128
NS = 16
NC = 2
K = 88
NCHUNK = 240
BI = 24
NB = NCHUNK // BI
E_PAD = NS * NCHUNK * K
N_PAD = 10240
ROWS_PER_SUB = N_PAD // NS


def _sc_agg_body(with_count, h0, h1, srcr, dstr, *rest):
    if with_count:
        (out0, out1, cnt_out, src_v, dst_v, r0, r1, r2, ones_v, ztile, zvec,
         acc, cntacc, sem_i, g0, g1, g2, s0, s1, s2) = rest
    else:
        (out0, out1, src_v, dst_v, r0, r1, r2, ztile, zvec,
         acc, sem_i, g0, g1, g2, s0, s1, s2) = rest
    c = lax.axis_index("c")
    s = lax.axis_index("s")
    R = (r0, r1, r2)
    G = (g0, g1, g2)
    S = (s0, s1, s2)

    for i in range(8):
        for j in range(C // 16):
            ztile[i, pl.ds(j * 16, 16)] = jnp.zeros((16,), jnp.float32)
    for j in range(ROWS_PER_SUB // 16):
        zvec[pl.ds(j * 16, 16)] = jnp.zeros((16,), jnp.float32)
    if with_count:
        for j in range(K // 16):
            ones_v[pl.ds(j * 16, 16)] = jnp.ones((16,), jnp.float32)
        if K % 16:
            ones_v[pl.ds(K - 16, 16)] = jnp.ones((16,), jnp.float32)

    def zero_body(i, _):
        pltpu.sync_copy(ztile, acc.at[pl.ds(s * ROWS_PER_SUB + i * 8, 8)])
        return 0
    lax.fori_loop(0, ROWS_PER_SUB // 8, zero_body, 0)
    if with_count:
        @pl.when(c == 0)
        def _():
            pltpu.sync_copy(zvec, cntacc.at[pl.ds(s * ROWS_PER_SUB, ROWS_PER_SUB)])

    plsc.subcore_barrier()

    def issue(row, buf, sem):
        @pl.when(c == 0)
        def _():
            pltpu.async_copy(h0.at[src_v.at[row]], buf, sem)
        @pl.when(c == 1)
        def _():
            pltpu.async_copy(h1.at[src_v.at[row]], buf, sem)

    def wait_g(buf, sem):
        pltpu.make_async_copy(h0.at[src_v.at[0]], buf, sem).wait()

    def scat(row, buf, sem):
        pltpu.async_copy(buf, acc.at[dst_v.at[row]], sem, add=True)

    def wait_s(buf, sem):
        pltpu.make_async_copy(buf, acc.at[dst_v.at[0]], sem).wait()

    pltpu.sync_copy(srcr.at[s, pl.ds(0, BI)], src_v.at[pl.ds(0, BI)])
    pltpu.sync_copy(dstr.at[s, pl.ds(0, BI)], dst_v.at[pl.ds(0, BI)])
    pltpu.async_copy(srcr.at[s, pl.ds(BI, BI)], src_v.at[pl.ds(BI, BI)], sem_i)
    pltpu.async_copy(dstr.at[s, pl.ds(BI, BI)], dst_v.at[pl.ds(BI, BI)], sem_i)
    issue(0, r0, g0)
    issue(1, r1, g1)

    def body(b, _):
        base = b * BI
        for t in range(BI):
            k = base + t
            row = ((b % 2) * BI + t)
            i3 = t % 3
            wait_g(R[i3], G[i3])
            scat(row, R[i3], S[i3])
            if with_count:
                @pl.when(c == 0)
                def _():
                    pltpu.sync_copy(ones_v, cntacc.at[dst_v.at[row]], add=True)
            if t == 1:
                @pl.when(jnp.logical_and(b >= 1, b + 1 < NB))
                def _():
                    nslot = ((b + 1) % 2) * BI
                    pltpu.async_copy(srcr.at[s, pl.ds((b + 1) * BI, BI)],
                                     src_v.at[pl.ds(nslot, BI)], sem_i)
                    pltpu.async_copy(dstr.at[s, pl.ds((b + 1) * BI, BI)],
                                     dst_v.at[pl.ds(nslot, BI)], sem_i)
            if t == BI - 2:
                @pl.when(b + 1 < NB)
                def _():
                    pltpu.make_async_copy(srcr.at[s, pl.ds(0, BI)],
                                          src_v.at[pl.ds(0, BI)], sem_i).wait()
                    pltpu.make_async_copy(dstr.at[s, pl.ds(0, BI)],
                                          dst_v.at[pl.ds(0, BI)], sem_i).wait()
            j3 = (t + 2) % 3
            @pl.when(k + 2 < NCHUNK)
            def _():
                @pl.when(k >= 1)
                def _():
                    wait_s(R[j3], S[j3])
                nrow = ((b % 2) * BI + t + 2) % (2 * BI)
                issue(nrow, R[j3], G[j3])
        return 0
    lax.fori_loop(0, NB, body, 0)

    for i3 in range(3):
        wait_s(R[i3], S[i3])
    plsc.subcore_barrier()
    @pl.when(c == 0)
    def _():
        pltpu.sync_copy(acc.at[pl.ds(s * ROWS_PER_SUB, ROWS_PER_SUB)],
                        out0.at[pl.ds(s * ROWS_PER_SUB, ROWS_PER_SUB)])
        if with_count:
            pltpu.sync_copy(cntacc.at[pl.ds(s * ROWS_PER_SUB, ROWS_PER_SUB)],
                            cnt_out.at[pl.ds(s * ROWS_PER_SUB, ROWS_PER_SUB)])
    @pl.when(c == 1)
    def _():
        pltpu.sync_copy(acc.at[pl.ds(s * ROWS_PER_SUB, ROWS_PER_SUB)],
                        out1.at[pl.ds(s * ROWS_PER_SUB, ROWS_PER_SUB)])


@functools.cache
def _make_sc_agg(with_count):
    mesh = plsc.VectorSubcoreMesh(core_axis_name="c", subcore_axis_name="s")
    out_type = [jax.ShapeDtypeStruct((N_PAD, C), jnp.float32),
                jax.ShapeDtypeStruct((N_PAD, C), jnp.float32)]
    scratch = [
        pltpu.VMEM((2 * BI, K), jnp.int32),
        pltpu.VMEM((2 * BI, K), jnp.int32),
        pltpu.VMEM((K, C), jnp.float32),
        pltpu.VMEM((K, C), jnp.float32),
        pltpu.VMEM((K, C), jnp.float32),
    ]
    if with_count:
        out_type.append(jax.ShapeDtypeStruct((N_PAD,), jnp.float32))
        scratch.append(pltpu.VMEM((K,), jnp.float32))
    scratch.append(pltpu.VMEM((8, C), jnp.float32))
    scratch.append(pltpu.VMEM((ROWS_PER_SUB,), jnp.float32))
    scratch.append(pltpu.VMEM_SHARED((N_PAD, C), jnp.float32))
    if with_count:
        scratch.append(pltpu.VMEM_SHARED((N_PAD,), jnp.float32))
    scratch.append(pltpu.SemaphoreType.DMA)
    scratch.extend([pltpu.SemaphoreType.DMA] * 3)
    scratch.extend([pltpu.SemaphoreType.DMA] * 3)
    return pl.kernel(
        functools.partial(_sc_agg_body, with_count),
        out_type=tuple(out_type),
        mesh=mesh,
        scratch_types=tuple(scratch),
    )


def _segment_pair(h0, h1, srcr, dstr, with_count):
    if with_count:
        a0, a1, cnt = _make_sc_agg(True)(h0, h1, srcr, dstr)
        return a0[:N], a1[:N], cnt[:N].reshape(N, 1)
    a0, a1 = _make_sc_agg(False)(h0, h1, srcr, dstr)
    return a0[:N], a1[:N]


BLK = 1000
GRID = N // BLK
_EPS = 1e-5


def _colstats(st_ref, y, idx, row):
    @pl.when(idx == 0)
    def _():
        st_ref[...] = jnp.zeros(st_ref.shape, st_ref.dtype)
    del row
    st_ref[0:1, :] += jnp.sum(y, axis=0, keepdims=True)
    st_ref[1:2, :] += jnp.sum(y * y, axis=0, keepdims=True)


def _norm(z, st, g, b):
    mu = st[0:1, :] * (1.0 / N)
    var = st[1:2, :] * (1.0 / N) - mu * mu
    inv = lax.rsqrt(var + _EPS)
    return (z - mu) * (inv * g) + b


def _stem_kernel(x_ref, w_ref, y_ref, st_ref):
    i = pl.program_id(0)
    y = jnp.dot(x_ref[...], w_ref[...], preferred_element_type=jnp.float32)
    y_ref[...] = y
    _colstats(st_ref, y, i, 0)


def _proj0_kernel(y_ref, st_ref, g_ref, b_ref, w0_ref, w1_ref,
                  z0_ref, st0_ref, z1_ref, st1_ref):
    i = pl.program_id(0)
    s = _norm(y_ref[...], st_ref[...], g_ref[...], b_ref[...])
    z0 = jnp.dot(s, w0_ref[...], preferred_element_type=jnp.float32)
    z1 = jnp.dot(s, w1_ref[...], preferred_element_type=jnp.float32)
    z0_ref[...] = z0
    z1_ref[...] = z1
    _colstats(st0_ref, z0, i, 0)
    _colstats(st1_ref, z1, i, 0)


def _norm2_kernel(z0_ref, st0_ref, g0_ref, b0_ref, z1_ref, st1_ref, g1_ref, b1_ref,
                  o0_ref, o1_ref):
    o0_ref[...] = jnp.maximum(_norm(z0_ref[...], st0_ref[...], g0_ref[...], b0_ref[...]), 0.0)
    o1_ref[...] = jnp.maximum(_norm(z1_ref[...], st1_ref[...], g1_ref[...], b1_ref[...]), 0.0)


def _cell0_proj1_kernel(s0_ref, a0_ref, s1_ref, a1_ref, cnt_ref,
                        wo0_ref, wo1_ref, wo2_ref, y_ref, sty_ref, gs_ref, bs_ref,
                        wp0_ref, wp1a_ref, wp1b_ref,
                        z0_ref, st0_ref, z1_ref, st1_ref):
    i = pl.program_id(0)
    rc = 1.0 / jnp.maximum(cnt_ref[...], 1.0)
    t0 = s0_ref[...] + a0_ref[...] * rc
    t1 = s1_ref[...] + a1_ref[...] * rc
    s2 = (jnp.maximum(jnp.dot(t0, wo0_ref[...], preferred_element_type=jnp.float32), 0.0)
          + jnp.maximum(jnp.dot(t1, wo1_ref[...], preferred_element_type=jnp.float32), 0.0))
    s3 = jnp.maximum(jnp.dot(t1, wo2_ref[...], preferred_element_type=jnp.float32), 0.0) + s2
    s = _norm(y_ref[...], sty_ref[...], gs_ref[...], bs_ref[...])
    z0 = jnp.dot(s, wp0_ref[...], preferred_element_type=jnp.float32)
    z1 = (jnp.dot(s2, wp1a_ref[...], preferred_element_type=jnp.float32)
          + jnp.dot(s3, wp1b_ref[...], preferred_element_type=jnp.float32))
    z0_ref[...] = z0
    z1_ref[...] = z1
    _colstats(st0_ref, z0, i, 0)
    _colstats(st1_ref, z1, i, 0)


def _cell1_cls_kernel(s0_ref, a0_ref, s1_ref, a1_ref, cnt_ref,
                      wo0_ref, wo1_ref, wo2_ref, wc0_ref, wca_ref, wcb_ref, bc_ref,
                      out_ref):
    rc = 1.0 / jnp.maximum(cnt_ref[...], 1.0)
    t0 = s0_ref[...] + a0_ref[...] * rc
    t1 = s1_ref[...] + a1_ref[...] * rc
    s2 = (jnp.maximum(jnp.dot(t0, wo0_ref[...], preferred_element_type=jnp.float32), 0.0)
          + jnp.maximum(jnp.dot(t1, wo1_ref[...], preferred_element_type=jnp.float32), 0.0))
    s3 = jnp.maximum(jnp.dot(t1, wo2_ref[...], preferred_element_type=jnp.float32), 0.0) + s2
    pooled = (jnp.sum(s2, axis=1, keepdims=True)
              + jnp.sum(s3, axis=1, keepdims=True)) * (1.0 / (2 * C))
    out_ref[...] = (jnp.dot(s2, wca_ref[...], preferred_element_type=jnp.float32)
                    + jnp.dot(s3, wcb_ref[...], preferred_element_type=jnp.float32)
                    + pooled * wc0_ref[...] + bc_ref[...])


def _row_spec(d):
    return pl.BlockSpec((BLK, d), lambda i: (i, 0))


def _full_spec(a, b):
    return pl.BlockSpec((a, b), lambda i: (0, 0))


def _f32(*shape):
    return jax.ShapeDtypeStruct(shape, jnp.float32)


def _stem(x, w):
    return pl.pallas_call(
        _stem_kernel, grid=(GRID,),
        in_specs=[_row_spec(x.shape[1]), _full_spec(*w.shape)],
        out_specs=[_row_spec(w.shape[1]), _full_spec(8, w.shape[1])],
        out_shape=[_f32(N, w.shape[1]), _f32(8, w.shape[1])],
    )(x, w)


def _proj0(y, st, g, b, w0, w1):
    d = y.shape[1]
    return pl.pallas_call(
        _proj0_kernel, grid=(GRID,),
        in_specs=[_row_spec(d), _full_spec(8, d), _full_spec(1, d), _full_spec(1, d),
                  _full_spec(d, C), _full_spec(d, C)],
        out_specs=[_row_spec(C), _full_spec(8, C),
                   _row_spec(C), _full_spec(8, C)],
        out_shape=[_f32(N, C), _f32(8, C), _f32(N, C), _f32(8, C)],
    )(y, st, g, b, w0, w1)


def _norm2(z0, st0, g0, b0, z1, st1, g1, b1):
    return pl.pallas_call(
        _norm2_kernel, grid=(GRID,),
        in_specs=[_row_spec(C), _full_spec(8, C), _full_spec(1, C), _full_spec(1, C),
                  _row_spec(C), _full_spec(8, C), _full_spec(1, C), _full_spec(1, C)],
        out_specs=[_row_spec(C), _row_spec(C)],
        out_shape=[_f32(N, C), _f32(N, C)],
    )(z0, st0, g0, b0, z1, st1, g1, b1)


def _cell0_proj1(s0, a0, s1, a1, cnt, wo0, wo1, wo2, y, sty, gs, bs, wp0, wp1a, wp1b):
    d = y.shape[1]
    return pl.pallas_call(
        _cell0_proj1_kernel, grid=(GRID,),
        in_specs=[_row_spec(C), _row_spec(C), _row_spec(C), _row_spec(C),
                  _row_spec(1),
                  _full_spec(C, C), _full_spec(C, C), _full_spec(C, C),
                  _row_spec(d), _full_spec(8, d), _full_spec(1, d), _full_spec(1, d),
                  _full_spec(d, C), _full_spec(C, C), _full_spec(C, C)],
        out_specs=[_row_spec(C), _full_spec(8, C), _row_spec(C), _full_spec(8, C)],
        out_shape=[_f32(N, C), _f32(8, C), _f32(N, C), _f32(8, C)],
    )(s0, a0, s1, a1, cnt, wo0, wo1, wo2, y, sty, gs, bs, wp0, wp1a, wp1b)


def _cell1_cls(s0, a0, s1, a1, cnt, wo0, wo1, wo2, wc0, wca, wcb, bc):
    nc = wca.shape[1]
    return pl.pallas_call(
        _cell1_cls_kernel, grid=(GRID,),
        in_specs=[_row_spec(C), _row_spec(C), _row_spec(C), _row_spec(C),
                  _row_spec(1),
                  _full_spec(C, C), _full_spec(C, C), _full_spec(C, C),
                  _full_spec(1, nc), _full_spec(C, nc), _full_spec(C, nc),
                  _full_spec(1, nc)],
        out_specs=_row_spec(nc),
        out_shape=_f32(N, nc),
    )(s0, a0, s1, a1, cnt, wo0, wo1, wo2, wc0, wca, wcb, bc)


def kernel(x, W_stem, g_stem, b_stem, Wp0_0, gp0_0, bp0_0, Wp1_0, gp1_0, bp1_0,
           Wo0_0, Wo1_0, Wo2_0, Wp0_1, gp0_1, bp0_1, Wp1_1, gp1_1, bp1_1,
           Wo0_1, Wo1_1, Wo2_1, Wc, bc, edge_index):
    src = edge_index[0]
    dst = edge_index[1]
    pad = E_PAD - E
    fidx = jnp.arange(pad, dtype=jnp.int32)
    srcr = jnp.concatenate([src, (fidx * 37) % N]).reshape(NS, NCHUNK, K)
    dstr = jnp.concatenate([dst, N + fidx % (N_PAD - N)]).reshape(NS, NCHUNK, K)

    r = lambda v: v.reshape(1, -1)

    y, sty = _stem(x, W_stem)
    z0, st0, z1, st1 = _proj0(y, sty, r(g_stem), r(b_stem), Wp0_0, Wp1_0)
    s0p, s1p = _norm2(z0, st0, r(gp0_0), r(bp0_0), z1, st1, r(gp1_0), r(bp1_0))
    a0, a1, cnt = _segment_pair(s0p, s1p, srcr, dstr, True)
    z0b, st0b, z1b, st1b = _cell0_proj1(
        s0p, a0, s1p, a1, cnt, Wo0_0, Wo1_0, Wo2_0, y, sty, r(g_stem), r(b_stem),
        Wp0_1, Wp1_1[:C], Wp1_1[C:])
    s0q, s1q = _norm2(z0b, st0b, r(gp0_1), r(bp0_1), z1b, st1b, r(gp1_1), r(bp1_1))
    b0, b1 = _segment_pair(s0q, s1q, srcr, dstr, False)
    logits = _cell1_cls(
        s0q, b0, s1q, b1, cnt, Wo0_1, Wo1_1, Wo2_1,
        r(Wc[0]), Wc[1:1 + C], Wc[1 + C:], r(bc))
    return logits


# speedup vs baseline: 8.0337x; 1.0089x over previous
import functools

import jax
import jax.numpy as jnp
from jax import lax
from jax.experimental import pallas as pl
from jax.experimental.pallas import tpu as pltpu
from jax.experimental.pallas import tpu_sc as plsc

N = 10000
E = 320000
C = 128
NS = 16
NC = 2
K = 88
NCHUNK = 240
BI = 24
NB = NCHUNK // BI
E_PAD = NS * NCHUNK * K
N_PAD = 10240
ROWS_PER_SUB = N_PAD // NS


def _sc_agg_body(with_count, h0, h1, srcr, dstr, *rest):
    if with_count:
        (out0, out1, cnt_out, src_v, dst_v, r0, r1, r2, ones_v, ztile, zvec,
         acc, cntacc, sem_i, g0, g1, g2, s0, s1, s2) = rest
    else:
        (out0, out1, src_v, dst_v, r0, r1, r2, ztile, zvec,
         acc, sem_i, g0, g1, g2, s0, s1, s2) = rest
    c = lax.axis_index("c")
    s = lax.axis_index("s")
    R = (r0, r1, r2)
    G = (g0, g1, g2)
    S = (s0, s1, s2)

    for i in range(8):
        for j in range(C // 16):
            ztile[i, pl.ds(j * 16, 16)] = jnp.zeros((16,), jnp.float32)
    for j in range(ROWS_PER_SUB // 16):
        zvec[pl.ds(j * 16, 16)] = jnp.zeros((16,), jnp.float32)
    if with_count:
        for j in range(K // 16):
            ones_v[pl.ds(j * 16, 16)] = jnp.ones((16,), jnp.float32)
        if K % 16:
            ones_v[pl.ds(K - 16, 16)] = jnp.ones((16,), jnp.float32)

    def zero_body(i, _):
        pltpu.sync_copy(ztile, acc.at[pl.ds(s * ROWS_PER_SUB + i * 8, 8)])
        return 0
    lax.fori_loop(0, ROWS_PER_SUB // 8, zero_body, 0)
    if with_count:
        @pl.when(c == 0)
        def _():
            pltpu.sync_copy(zvec, cntacc.at[pl.ds(s * ROWS_PER_SUB, ROWS_PER_SUB)])

    plsc.subcore_barrier()

    def issue(row, buf, sem):
        @pl.when(c == 0)
        def _():
            pltpu.async_copy(h0.at[src_v.at[row]], buf, sem)
        @pl.when(c == 1)
        def _():
            pltpu.async_copy(h1.at[src_v.at[row]], buf, sem)

    def wait_g(buf, sem):
        pltpu.make_async_copy(h0.at[src_v.at[0]], buf, sem).wait()

    def scat(row, buf, sem):
        pltpu.async_copy(buf, acc.at[dst_v.at[row]], sem, add=True)

    def wait_s(buf, sem):
        pltpu.make_async_copy(buf, acc.at[dst_v.at[0]], sem).wait()

    pltpu.sync_copy(srcr.at[s, pl.ds(0, BI)], src_v.at[pl.ds(0, BI)])
    pltpu.sync_copy(dstr.at[s, pl.ds(0, BI)], dst_v.at[pl.ds(0, BI)])
    pltpu.async_copy(srcr.at[s, pl.ds(BI, BI)], src_v.at[pl.ds(BI, BI)], sem_i)
    pltpu.async_copy(dstr.at[s, pl.ds(BI, BI)], dst_v.at[pl.ds(BI, BI)], sem_i)
    issue(0, r0, g0)
    issue(1, r1, g1)

    def body(b, _):
        base = b * BI
        for t in range(BI):
            k = base + t
            row = ((b % 2) * BI + t)
            i3 = t % 3
            wait_g(R[i3], G[i3])
            scat(row, R[i3], S[i3])
            if with_count:
                @pl.when(c == 0)
                def _():
                    pltpu.sync_copy(ones_v, cntacc.at[dst_v.at[row]], add=True)
            if t == 1:
                @pl.when(jnp.logical_and(b >= 1, b + 1 < NB))
                def _():
                    nslot = ((b + 1) % 2) * BI
                    pltpu.async_copy(srcr.at[s, pl.ds((b + 1) * BI, BI)],
                                     src_v.at[pl.ds(nslot, BI)], sem_i)
                    pltpu.async_copy(dstr.at[s, pl.ds((b + 1) * BI, BI)],
                                     dst_v.at[pl.ds(nslot, BI)], sem_i)
            if t == BI - 2:
                @pl.when(b + 1 < NB)
                def _():
                    pltpu.make_async_copy(srcr.at[s, pl.ds(0, BI)],
                                          src_v.at[pl.ds(0, BI)], sem_i).wait()
                    pltpu.make_async_copy(dstr.at[s, pl.ds(0, BI)],
                                          dst_v.at[pl.ds(0, BI)], sem_i).wait()
            j3 = (t + 2) % 3
            @pl.when(k + 2 < NCHUNK)
            def _():
                @pl.when(k >= 1)
                def _():
                    wait_s(R[j3], S[j3])
                nrow = ((b % 2) * BI + t + 2) % (2 * BI)
                issue(nrow, R[j3], G[j3])
        return 0
    lax.fori_loop(0, NB, body, 0)

    for i3 in range(3):
        wait_s(R[i3], S[i3])
    plsc.subcore_barrier()
    @pl.when(c == 0)
    def _():
        pltpu.sync_copy(acc.at[pl.ds(s * ROWS_PER_SUB, ROWS_PER_SUB)],
                        out0.at[pl.ds(s * ROWS_PER_SUB, ROWS_PER_SUB)])
        if with_count:
            pltpu.sync_copy(cntacc.at[pl.ds(s * ROWS_PER_SUB, ROWS_PER_SUB)],
                            cnt_out.at[pl.ds(s * ROWS_PER_SUB, ROWS_PER_SUB)])
    @pl.when(c == 1)
    def _():
        pltpu.sync_copy(acc.at[pl.ds(s * ROWS_PER_SUB, ROWS_PER_SUB)],
                        out1.at[pl.ds(s * ROWS_PER_SUB, ROWS_PER_SUB)])


@functools.cache
def _make_sc_agg(with_count):
    mesh = plsc.VectorSubcoreMesh(core_axis_name="c", subcore_axis_name="s")
    out_type = [jax.ShapeDtypeStruct((N_PAD, C), jnp.float32),
                jax.ShapeDtypeStruct((N_PAD, C), jnp.float32)]
    scratch = [
        pltpu.VMEM((2 * BI, K), jnp.int32),
        pltpu.VMEM((2 * BI, K), jnp.int32),
        pltpu.VMEM((K, C), jnp.float32),
        pltpu.VMEM((K, C), jnp.float32),
        pltpu.VMEM((K, C), jnp.float32),
    ]
    if with_count:
        out_type.append(jax.ShapeDtypeStruct((N_PAD,), jnp.float32))
        scratch.append(pltpu.VMEM((K,), jnp.float32))
    scratch.append(pltpu.VMEM((8, C), jnp.float32))
    scratch.append(pltpu.VMEM((ROWS_PER_SUB,), jnp.float32))
    scratch.append(pltpu.VMEM_SHARED((N_PAD, C), jnp.float32))
    if with_count:
        scratch.append(pltpu.VMEM_SHARED((N_PAD,), jnp.float32))
    scratch.append(pltpu.SemaphoreType.DMA)
    scratch.extend([pltpu.SemaphoreType.DMA] * 3)
    scratch.extend([pltpu.SemaphoreType.DMA] * 3)
    return pl.kernel(
        functools.partial(_sc_agg_body, with_count),
        out_type=tuple(out_type),
        mesh=mesh,
        scratch_types=tuple(scratch),
    )


def _segment_pair(h0, h1, srcr, dstr, with_count):
    if with_count:
        a0, a1, cnt = _make_sc_agg(True)(h0, h1, srcr, dstr)
        return a0[:N], a1[:N], cnt[:N].reshape(N, 1)
    a0, a1 = _make_sc_agg(False)(h0, h1, srcr, dstr)
    return a0[:N], a1[:N]


BLK = 1000
GRID = N // BLK
_EPS = 1e-5


def _colstats(st_ref, y, idx, row):
    @pl.when(idx == 0)
    def _():
        st_ref[...] = jnp.zeros(st_ref.shape, st_ref.dtype)
    del row
    st_ref[0:1, :] += jnp.sum(y, axis=0, keepdims=True)
    st_ref[1:2, :] += jnp.sum(y * y, axis=0, keepdims=True)


def _norm(z, st, g, b):
    mu = st[0:1, :] * (1.0 / N)
    var = st[1:2, :] * (1.0 / N) - mu * mu
    inv = lax.rsqrt(var + _EPS)
    return (z - mu) * (inv * g) + b


def _stem_kernel(x_ref, w_ref, y_ref, st_ref):
    i = pl.program_id(0)
    y = jnp.dot(x_ref[...], w_ref[...], preferred_element_type=jnp.float32)
    y_ref[...] = y
    _colstats(st_ref, y, i, 0)


def _proj0_kernel(y_ref, st_ref, g_ref, b_ref, w0_ref, w1_ref,
                  z0_ref, st0_ref, z1_ref, st1_ref):
    i = pl.program_id(0)
    s = _norm(y_ref[...], st_ref[...], g_ref[...], b_ref[...])
    z0 = jnp.dot(s, w0_ref[...], preferred_element_type=jnp.float32)
    z1 = jnp.dot(s, w1_ref[...], preferred_element_type=jnp.float32)
    z0_ref[...] = z0
    z1_ref[...] = z1
    _colstats(st0_ref, z0, i, 0)
    _colstats(st1_ref, z1, i, 0)


def _norm2_kernel(z0_ref, st0_ref, g0_ref, b0_ref, z1_ref, st1_ref, g1_ref, b1_ref,
                  o0_ref, o1_ref):
    o0_ref[...] = jnp.maximum(_norm(z0_ref[...], st0_ref[...], g0_ref[...], b0_ref[...]), 0.0)
    o1_ref[...] = jnp.maximum(_norm(z1_ref[...], st1_ref[...], g1_ref[...], b1_ref[...]), 0.0)


def _projpre_kernel(y_ref, sty_ref, g_ref, b_ref, w_ref, z_ref, stz_ref):
    i = pl.program_id(0)
    s = _norm(y_ref[...], sty_ref[...], g_ref[...], b_ref[...])
    z = jnp.dot(s, w_ref[...], preferred_element_type=jnp.float32)
    z_ref[...] = z
    _colstats(stz_ref, z, i, 0)


def _norm1_kernel(z_ref, st_ref, g_ref, b_ref, o_ref):
    o_ref[...] = jnp.maximum(_norm(z_ref[...], st_ref[...], g_ref[...], b_ref[...]), 0.0)


def _cell0_proj1_kernel(s0_ref, a0_ref, s1_ref, a1_ref, cnt_ref,
                        wo0_ref, wo1_ref, wo2_ref,
                        wp1a_ref, wp1b_ref,
                        z1_ref, st1_ref):
    i = pl.program_id(0)
    rc = 1.0 / jnp.maximum(cnt_ref[...], 1.0)
    t0 = s0_ref[...] + a0_ref[...] * rc
    t1 = s1_ref[...] + a1_ref[...] * rc
    s2 = (jnp.maximum(jnp.dot(t0, wo0_ref[...], preferred_element_type=jnp.float32), 0.0)
          + jnp.maximum(jnp.dot(t1, wo1_ref[...], preferred_element_type=jnp.float32), 0.0))
    s3 = jnp.maximum(jnp.dot(t1, wo2_ref[...], preferred_element_type=jnp.float32), 0.0) + s2
    z1 = (jnp.dot(s2, wp1a_ref[...], preferred_element_type=jnp.float32)
          + jnp.dot(s3, wp1b_ref[...], preferred_element_type=jnp.float32))
    z1_ref[...] = z1
    _colstats(st1_ref, z1, i, 0)


def _cell1_cls_kernel(s0_ref, a0_ref, s1_ref, a1_ref, cnt_ref,
                      wo0_ref, wo1_ref, wo2_ref, wc0_ref, wca_ref, wcb_ref, bc_ref,
                      out_ref):
    rc = 1.0 / jnp.maximum(cnt_ref[...], 1.0)
    t0 = s0_ref[...] + a0_ref[...] * rc
    t1 = s1_ref[...] + a1_ref[...] * rc
    s2 = (jnp.maximum(jnp.dot(t0, wo0_ref[...], preferred_element_type=jnp.float32), 0.0)
          + jnp.maximum(jnp.dot(t1, wo1_ref[...], preferred_element_type=jnp.float32), 0.0))
    s3 = jnp.maximum(jnp.dot(t1, wo2_ref[...], preferred_element_type=jnp.float32), 0.0) + s2
    pooled = (jnp.sum(s2, axis=1, keepdims=True)
              + jnp.sum(s3, axis=1, keepdims=True)) * (1.0 / (2 * C))
    out_ref[...] = (jnp.dot(s2, wca_ref[...], preferred_element_type=jnp.float32)
                    + jnp.dot(s3, wcb_ref[...], preferred_element_type=jnp.float32)
                    + pooled * wc0_ref[...] + bc_ref[...])


def _row_spec(d):
    return pl.BlockSpec((BLK, d), lambda i: (i, 0))


def _full_spec(a, b):
    return pl.BlockSpec((a, b), lambda i: (0, 0))


def _f32(*shape):
    return jax.ShapeDtypeStruct(shape, jnp.float32)


def _stem(x, w):
    return pl.pallas_call(
        _stem_kernel, grid=(GRID,),
        in_specs=[_row_spec(x.shape[1]), _full_spec(*w.shape)],
        out_specs=[_row_spec(w.shape[1]), _full_spec(8, w.shape[1])],
        out_shape=[_f32(N, w.shape[1]), _f32(8, w.shape[1])],
    )(x, w)


def _proj0(y, st, g, b, w0, w1):
    d = y.shape[1]
    return pl.pallas_call(
        _proj0_kernel, grid=(GRID,),
        in_specs=[_row_spec(d), _full_spec(8, d), _full_spec(1, d), _full_spec(1, d),
                  _full_spec(d, C), _full_spec(d, C)],
        out_specs=[_row_spec(C), _full_spec(8, C),
                   _row_spec(C), _full_spec(8, C)],
        out_shape=[_f32(N, C), _f32(8, C), _f32(N, C), _f32(8, C)],
    )(y, st, g, b, w0, w1)


def _norm2(z0, st0, g0, b0, z1, st1, g1, b1):
    return pl.pallas_call(
        _norm2_kernel, grid=(GRID,),
        in_specs=[_row_spec(C), _full_spec(8, C), _full_spec(1, C), _full_spec(1, C),
                  _row_spec(C), _full_spec(8, C), _full_spec(1, C), _full_spec(1, C)],
        out_specs=[_row_spec(C), _row_spec(C)],
        out_shape=[_f32(N, C), _f32(N, C)],
    )(z0, st0, g0, b0, z1, st1, g1, b1)


def _projpre(y, sty, g, b, w):
    d = y.shape[1]
    return pl.pallas_call(
        _projpre_kernel, grid=(GRID,),
        in_specs=[_row_spec(d), _full_spec(8, d), _full_spec(1, d), _full_spec(1, d),
                  _full_spec(d, C)],
        out_specs=[_row_spec(C), _full_spec(8, C)],
        out_shape=[_f32(N, C), _f32(8, C)],
    )(y, sty, g, b, w)


def _norm1(z, st, g, b):
    return pl.pallas_call(
        _norm1_kernel, grid=(GRID,),
        in_specs=[_row_spec(C), _full_spec(8, C), _full_spec(1, C), _full_spec(1, C)],
        out_specs=_row_spec(C),
        out_shape=_f32(N, C),
    )(z, st, g, b)


def _cell0_proj1(s0, a0, s1, a1, cnt, wo0, wo1, wo2, wp1a, wp1b):
    return pl.pallas_call(
        _cell0_proj1_kernel, grid=(GRID,),
        in_specs=[_row_spec(C), _row_spec(C), _row_spec(C), _row_spec(C),
                  _row_spec(1),
                  _full_spec(C, C), _full_spec(C, C), _full_spec(C, C),
                  _full_spec(C, C), _full_spec(C, C)],
        out_specs=[_row_spec(C), _full_spec(8, C)],
        out_shape=[_f32(N, C), _f32(8, C)],
    )(s0, a0, s1, a1, cnt, wo0, wo1, wo2, wp1a, wp1b)


def _cell1_cls(s0, a0, s1, a1, cnt, wo0, wo1, wo2, wc0, wca, wcb, bc):
    nc = wca.shape[1]
    return pl.pallas_call(
        _cell1_cls_kernel, grid=(GRID,),
        in_specs=[_row_spec(C), _row_spec(C), _row_spec(C), _row_spec(C),
                  _row_spec(1),
                  _full_spec(C, C), _full_spec(C, C), _full_spec(C, C),
                  _full_spec(1, nc), _full_spec(C, nc), _full_spec(C, nc),
                  _full_spec(1, nc)],
        out_specs=_row_spec(nc),
        out_shape=_f32(N, nc),
    )(s0, a0, s1, a1, cnt, wo0, wo1, wo2, wc0, wca, wcb, bc)


def kernel(x, W_stem, g_stem, b_stem, Wp0_0, gp0_0, bp0_0, Wp1_0, gp1_0, bp1_0,
           Wo0_0, Wo1_0, Wo2_0, Wp0_1, gp0_1, bp0_1, Wp1_1, gp1_1, bp1_1,
           Wo0_1, Wo1_1, Wo2_1, Wc, bc, edge_index):
    src = edge_index[0]
    dst = edge_index[1]
    pad = E_PAD - E
    fidx = jnp.arange(pad, dtype=jnp.int32)
    srcr = jnp.concatenate([src, (fidx * 37) % N]).reshape(NS, NCHUNK, K)
    dstr = jnp.concatenate([dst, N + fidx % (N_PAD - N)]).reshape(NS, NCHUNK, K)

    r = lambda v: v.reshape(1, -1)

    y, sty = _stem(x, W_stem)
    z0, st0, z1, st1 = _proj0(y, sty, r(g_stem), r(b_stem), Wp0_0, Wp1_0)
    s0p, s1p = _norm2(z0, st0, r(gp0_0), r(bp0_0), z1, st1, r(gp1_0), r(bp1_0))
    z0b, st0b = _projpre(y, sty, r(g_stem), r(b_stem), Wp0_1)
    a0, a1, cnt = _segment_pair(s0p, s1p, srcr, dstr, True)
    s0q = _norm1(z0b, st0b, r(gp0_1), r(bp0_1))
    z1b, st1b = _cell0_proj1(
        s0p, a0, s1p, a1, cnt, Wo0_0, Wo1_0, Wo2_0, Wp1_1[:C], Wp1_1[C:])
    s1q = _norm1(z1b, st1b, r(gp1_1), r(bp1_1))
    b0, b1 = _segment_pair(s0q, s1q, srcr, dstr, False)
    logits = _cell1_cls(
        s0q, b0, s1q, b1, cnt, Wo0_1, Wo1_1, Wo2_1,
        r(Wc[0]), Wc[1:1 + C], Wc[1 + C:], r(bc))
    return logits
